# Initial kernel scaffold; baseline (speedup 1.0000x reference)
import jax
import jax.numpy as jnp
from jax.experimental import pallas as pl


def kernel(x, edge_index, W1, b1, W2, b2, W3, b3):
    raise NotImplementedError("write your pallas kernel here")


# speedup vs baseline: 13.8847x; 13.8847x over previous
import functools

import jax
import jax.numpy as jnp
from jax import lax
from jax.experimental import pallas as pl
from jax.experimental.pallas import tpu as pltpu
from jax.experimental.pallas import tpu_sc as plsc

_NC, _NS = 2, 16
_NW = _NC * _NS
_K = 80
_R = 1024


def _sc_degree(dst_i32, z_hbm, tpad):
    e = dst_i32.shape[0]
    ew = e // _NW
    nch = ew // _K
    rows = tpad // _NS
    mesh = plsc.VectorSubcoreMesh(core_axis_name="c", subcore_axis_name="s", num_cores=_NC, num_subcores=_NS)

    @functools.partial(
        pl.kernel,
        out_type=jax.ShapeDtypeStruct((_NC, tpad), jnp.float32),
        mesh=mesh,
        compiler_params=pltpu.CompilerParams(use_tc_tiling_on_sc=False),
        scratch_types=[
            pltpu.VMEM((_K,), jnp.int32),
            pltpu.VMEM((_K,), jnp.float32),
            pltpu.VMEM_SHARED((tpad,), jnp.float32),
        ],
    )
    def deg_kernel(dst_hbm, zz_hbm, out_hbm, idx_v, ones_v, acc_sh):
        c = lax.axis_index("c")
        s = lax.axis_index("s")
        w = s * _NC + c
        base = s * rows
        for i in range(_K // 16):
            ones_v[pl.ds(i * 16, 16)] = jnp.full((16,), 1.0, jnp.float32)
        pltpu.sync_copy(zz_hbm.at[pl.ds(base, rows)], acc_sh.at[pl.ds(base, rows)])
        plsc.subcore_barrier()
        ebase = w * ew

        def body(j, carry):
            off = ebase + j * _K
            pltpu.sync_copy(dst_hbm.at[pl.ds(off, _K)], idx_v)
            pltpu.sync_copy(ones_v, acc_sh.at[idx_v], add=True)
            return carry

        lax.fori_loop(0, nch, body, 0)
        plsc.subcore_barrier()
        pltpu.sync_copy(acc_sh.at[pl.ds(base, rows)], out_hbm.at[c, pl.ds(base, rows)])

    return deg_kernel(dst_i32, z_hbm)


def _sc_scatter(src_i32, dst_i32, p, z_hbm):
    e = src_i32.shape[0]
    tpad, d = p.shape
    ew = e // _NW
    nch = ew // _K
    rows = tpad // _NS
    mesh = plsc.VectorSubcoreMesh(core_axis_name="c", subcore_axis_name="s", num_cores=_NC, num_subcores=_NS)

    @functools.partial(
        pl.kernel,
        out_type=jax.ShapeDtypeStruct((_NC, tpad, d), jnp.float32),
        mesh=mesh,
        compiler_params=pltpu.CompilerParams(use_tc_tiling_on_sc=False),
        scratch_types=[
            pltpu.VMEM((_K,), jnp.int32),
            pltpu.VMEM((_K,), jnp.int32),
            pltpu.VMEM((_K, d), jnp.float32),
            pltpu.VMEM_SHARED((tpad, d), jnp.float32),
            pltpu.SemaphoreType.DMA,
        ],
    )
    def scat_kernel(src_hbm, dst_hbm, p_hbm, zz_hbm, out_hbm,
                    isrc_v, idst_v, rows_v, acc_sh, sem):
        c = lax.axis_index("c")
        s = lax.axis_index("s")
        w = s * _NC + c
        base = s * rows
        pltpu.sync_copy(zz_hbm.at[pl.ds(base, rows)], acc_sh.at[pl.ds(base, rows)])
        plsc.subcore_barrier()
        ebase = w * ew

        def body(j, carry):
            off = ebase + j * _K
            pltpu.sync_copy(src_hbm.at[pl.ds(off, _K)], isrc_v)
            pltpu.sync_copy(dst_hbm.at[pl.ds(off, _K)], idst_v)
            pltpu.async_copy(p_hbm.at[isrc_v], rows_v, sem).wait()
            pltpu.sync_copy(rows_v, acc_sh.at[idst_v], add=True)
            return carry

        lax.fori_loop(0, nch, body, 0)
        plsc.subcore_barrier()
        pltpu.sync_copy(acc_sh.at[pl.ds(base, rows)],
                        out_hbm.at[c, pl.ds(base, rows)])

    return scat_kernel(src_i32, dst_i32, p, z_hbm)


def _tc_first(x, w1, dg0, dg1):
    n, d_in = x.shape
    d_out = w1.shape[1]

    def body(x_ref, w_ref, d0_ref, d1_ref, p_ref, dis_ref):
        deg = d0_ref[...] + d1_ref[...] + 1.0
        dis = lax.rsqrt(deg)
        h = jnp.dot(x_ref[...], w_ref[...], preferred_element_type=jnp.float32)
        p_ref[...] = h * dis
        dis_ref[...] = dis

    return pl.pallas_call(
        body,
        grid=(n // _R,),
        in_specs=[
            pl.BlockSpec((_R, d_in), lambda i: (i, 0)),
            pl.BlockSpec((d_in, d_out), lambda i: (0, 0)),
            pl.BlockSpec((_R, 1), lambda i: (i, 0)),
            pl.BlockSpec((_R, 1), lambda i: (i, 0)),
        ],
        out_specs=[
            pl.BlockSpec((_R, d_out), lambda i: (i, 0)),
            pl.BlockSpec((_R, 1), lambda i: (i, 0)),
        ],
        out_shape=[
            jax.ShapeDtypeStruct((n, d_out), jnp.float32),
            jax.ShapeDtypeStruct((n, 1), jnp.float32),
        ],
    )(x, w1, dg0, dg1)


def _tc_mid(p, a0, a1, dis, b, w, out_width):
    n, d = p.shape
    d2 = w.shape[1]

    def body(p_ref, a0_ref, a1_ref, dis_ref, b_ref, w_ref, o_ref):
        dis = dis_ref[...]
        hbar = (a0_ref[...] + a1_ref[...] + p_ref[...]) * dis + b_ref[...]
        h = jnp.maximum(hbar, 0.0)
        q = jnp.dot(h, w_ref[...], preferred_element_type=jnp.float32) * dis
        if out_width != d2:
            q = jnp.broadcast_to(q, (q.shape[0], out_width))
        o_ref[...] = q

    return pl.pallas_call(
        body,
        grid=(n // _R,),
        in_specs=[
            pl.BlockSpec((_R, d), lambda i: (i, 0)),
            pl.BlockSpec((_R, d), lambda i: (i, 0)),
            pl.BlockSpec((_R, d), lambda i: (i, 0)),
            pl.BlockSpec((_R, 1), lambda i: (i, 0)),
            pl.BlockSpec((1, d), lambda i: (0, 0)),
            pl.BlockSpec((d, d2), lambda i: (0, 0)),
        ],
        out_specs=pl.BlockSpec((_R, out_width), lambda i: (i, 0)),
        out_shape=jax.ShapeDtypeStruct((n, out_width), jnp.float32),
    )(p, a0, a1, dis, b, w)


def _tc_last(p16, a0, a1, dis, b3):
    n, d = p16.shape

    def body(p_ref, a0_ref, a1_ref, dis_ref, b_ref, o_ref):
        acc = a0_ref[...][:, :1] + a1_ref[...][:, :1] + p_ref[...][:, :1]
        v = acc * dis_ref[...] + b_ref[...]
        o_ref[...] = jax.nn.sigmoid(v)

    return pl.pallas_call(
        body,
        grid=(n // _R,),
        in_specs=[
            pl.BlockSpec((_R, d), lambda i: (i, 0)),
            pl.BlockSpec((_R, d), lambda i: (i, 0)),
            pl.BlockSpec((_R, d), lambda i: (i, 0)),
            pl.BlockSpec((_R, 1), lambda i: (i, 0)),
            pl.BlockSpec((1, 1), lambda i: (0, 0)),
        ],
        out_specs=pl.BlockSpec((_R, 1), lambda i: (i, 0)),
        out_shape=jax.ShapeDtypeStruct((n, 1), jnp.float32),
    )(p16, a0, a1, dis, b3)


def kernel(x, edge_index, W1, b1, W2, b2, W3, b3):
    n, d_feat = x.shape
    e = edge_index.shape[1]
    tpad = ((n + _R - 1) // _R) * _R
    assert tpad % (_NS * 8) == 0 and e % (_NW * _K) == 0

    src = edge_index[0].astype(jnp.int32)
    dst = edge_index[1].astype(jnp.int32)
    x_pad = jnp.zeros((tpad, d_feat), jnp.float32).at[:n].set(x)
    z1 = jnp.zeros((tpad,), jnp.float32)
    z64 = jnp.zeros((tpad, W1.shape[1]), jnp.float32)
    z32 = jnp.zeros((tpad, W2.shape[1]), jnp.float32)
    z16 = jnp.zeros((tpad, 16), jnp.float32)

    degp = _sc_degree(dst, z1, tpad)
    dg0 = degp[0].reshape(tpad, 1)
    dg1 = degp[1].reshape(tpad, 1)
    p1, dis = _tc_first(x_pad, W1, dg0, dg1)
    a = _sc_scatter(src, dst, p1, z64)
    p2 = _tc_mid(p1, a[0], a[1], dis, b1.reshape(1, -1), W2, W2.shape[1])
    a = _sc_scatter(src, dst, p2, z32)
    p3 = _tc_mid(p2, a[0], a[1], dis, b2.reshape(1, -1), W3, 16)
    a = _sc_scatter(src, dst, p3, z16)
    out = _tc_last(p3, a[0], a[1], dis, b3.reshape(1, 1))
    return out[:n]


# speedup vs baseline: 38.4833x; 2.7716x over previous
import functools

import jax
import jax.numpy as jnp
from jax import lax
from jax.experimental import pallas as pl
from jax.experimental.pallas import tpu as pltpu
from jax.experimental.pallas import tpu_sc as plsc

_NC, _NS = 2, 16
_NW = _NC * _NS
_K = 80
_NB = 5
_R = 1024


def _sc_degree(dst2d, z_hbm, tpad):
    nw, nch, k = dst2d.shape
    rows = tpad // _NS
    nwave = nch // _NB
    mesh = plsc.VectorSubcoreMesh(core_axis_name="c", subcore_axis_name="s", num_cores=_NC, num_subcores=_NS)

    @functools.partial(
        pl.kernel,
        out_type=jax.ShapeDtypeStruct((_NC, tpad), jnp.float32),
        mesh=mesh,
        compiler_params=pltpu.CompilerParams(use_tc_tiling_on_sc=False),
        scratch_types=[
            pltpu.VMEM((nch, k), jnp.int32),
            pltpu.VMEM((k,), jnp.float32),
            pltpu.VMEM_SHARED((tpad,), jnp.float32),
            pltpu.SemaphoreType.DMA,
        ],
    )
    def deg_kernel(dst_hbm, zz_hbm, out_hbm, idst2, ones_v, acc_sh, ssem):
        c = lax.axis_index("c")
        s = lax.axis_index("s")
        w = s * _NC + c
        base = s * rows
        for i in range(k // 16):
            ones_v[pl.ds(i * 16, 16)] = jnp.full((16,), 1.0, jnp.float32)
        pltpu.sync_copy(zz_hbm.at[pl.ds(base, rows)], acc_sh.at[pl.ds(base, rows)])
        pltpu.sync_copy(dst_hbm.at[w], idst2)
        plsc.subcore_barrier()

        def wave(t, carry):
            j0 = t * _NB
            descs = [
                pltpu.async_copy(ones_v, acc_sh.at[idst2.at[j0 + b]], ssem, add=True)
                for b in range(_NB)
            ]
            for dsc in descs:
                dsc.wait()
            return carry

        lax.fori_loop(0, nwave, wave, 0)
        plsc.subcore_barrier()
        pltpu.sync_copy(acc_sh.at[pl.ds(base, rows)], out_hbm.at[c, pl.ds(base, rows)])

    return deg_kernel(dst2d, z_hbm)


def _sc_scatter(src2d, dst2d, p, z_hbm):
    nw, nch, k = src2d.shape
    tpad, d = p.shape
    rows = tpad // _NS
    nwave = nch // _NB
    mesh = plsc.VectorSubcoreMesh(core_axis_name="c", subcore_axis_name="s", num_cores=_NC, num_subcores=_NS)

    @functools.partial(
        pl.kernel,
        out_type=jax.ShapeDtypeStruct((_NC, tpad, d), jnp.float32),
        mesh=mesh,
        compiler_params=pltpu.CompilerParams(use_tc_tiling_on_sc=False),
        scratch_types=[
            pltpu.VMEM((nch, k), jnp.int32),
            pltpu.VMEM((nch, k), jnp.int32),
            pltpu.VMEM((_NB, k, d), jnp.float32),
            pltpu.VMEM_SHARED((tpad, d), jnp.float32),
        ] + [pltpu.SemaphoreType.DMA] * _NB + [pltpu.SemaphoreType.DMA],
    )
    def scat_kernel(src_hbm, dst_hbm, p_hbm, zz_hbm, out_hbm,
                    isrc2, idst2, rows_v, acc_sh, *sems):
        gsems = sems[:_NB]
        ssem = sems[_NB]
        c = lax.axis_index("c")
        s = lax.axis_index("s")
        w = s * _NC + c
        base = s * rows
        pltpu.sync_copy(zz_hbm.at[pl.ds(base, rows)], acc_sh.at[pl.ds(base, rows)])
        pltpu.sync_copy(src_hbm.at[w], isrc2)
        pltpu.sync_copy(dst_hbm.at[w], idst2)
        plsc.subcore_barrier()

        def wave(t, carry):
            j0 = t * _NB
            gds = [
                pltpu.async_copy(p_hbm.at[isrc2.at[j0 + b]], rows_v.at[b], gsems[b])
                for b in range(_NB)
            ]
            sds = []
            for b in range(_NB):
                gds[b].wait()
                sds.append(pltpu.async_copy(
                    rows_v.at[b], acc_sh.at[idst2.at[j0 + b]], ssem, add=True))
            for dsc in sds:
                dsc.wait()
            return carry

        lax.fori_loop(0, nwave, wave, 0)
        plsc.subcore_barrier()
        pltpu.sync_copy(acc_sh.at[pl.ds(base, rows)],
                        out_hbm.at[c, pl.ds(base, rows)])

    return scat_kernel(src2d, dst2d, p, z_hbm)


def _tc_first(x, w1, dg0, dg1):
    n, d_in = x.shape
    d_out = w1.shape[1]

    def body(x_ref, w_ref, d0_ref, d1_ref, p_ref, dis_ref):
        deg = d0_ref[...] + d1_ref[...] + 1.0
        dis = lax.rsqrt(deg)
        h = jnp.dot(x_ref[...], w_ref[...], preferred_element_type=jnp.float32)
        p_ref[...] = h * dis
        dis_ref[...] = dis

    return pl.pallas_call(
        body,
        grid=(n // _R,),
        in_specs=[
            pl.BlockSpec((_R, d_in), lambda i: (i, 0)),
            pl.BlockSpec((d_in, d_out), lambda i: (0, 0)),
            pl.BlockSpec((_R, 1), lambda i: (i, 0)),
            pl.BlockSpec((_R, 1), lambda i: (i, 0)),
        ],
        out_specs=[
            pl.BlockSpec((_R, d_out), lambda i: (i, 0)),
            pl.BlockSpec((_R, 1), lambda i: (i, 0)),
        ],
        out_shape=[
            jax.ShapeDtypeStruct((n, d_out), jnp.float32),
            jax.ShapeDtypeStruct((n, 1), jnp.float32),
        ],
    )(x, w1, dg0, dg1)


def _tc_mid(p, a0, a1, dis, b, w, out_width):
    n, d = p.shape
    d2 = w.shape[1]

    def body(p_ref, a0_ref, a1_ref, dis_ref, b_ref, w_ref, o_ref):
        dis = dis_ref[...]
        hbar = (a0_ref[...] + a1_ref[...] + p_ref[...]) * dis + b_ref[...]
        h = jnp.maximum(hbar, 0.0)
        q = jnp.dot(h, w_ref[...], preferred_element_type=jnp.float32) * dis
        if out_width != d2:
            q = jnp.broadcast_to(q, (q.shape[0], out_width))
        o_ref[...] = q

    return pl.pallas_call(
        body,
        grid=(n // _R,),
        in_specs=[
            pl.BlockSpec((_R, d), lambda i: (i, 0)),
            pl.BlockSpec((_R, d), lambda i: (i, 0)),
            pl.BlockSpec((_R, d), lambda i: (i, 0)),
            pl.BlockSpec((_R, 1), lambda i: (i, 0)),
            pl.BlockSpec((1, d), lambda i: (0, 0)),
            pl.BlockSpec((d, d2), lambda i: (0, 0)),
        ],
        out_specs=pl.BlockSpec((_R, out_width), lambda i: (i, 0)),
        out_shape=jax.ShapeDtypeStruct((n, out_width), jnp.float32),
    )(p, a0, a1, dis, b, w)


def _tc_last(p16, a0, a1, dis, b3):
    n, d = p16.shape

    def body(p_ref, a0_ref, a1_ref, dis_ref, b_ref, o_ref):
        acc = a0_ref[...][:, :1] + a1_ref[...][:, :1] + p_ref[...][:, :1]
        v = acc * dis_ref[...] + b_ref[...]
        o_ref[...] = jax.nn.sigmoid(v)

    return pl.pallas_call(
        body,
        grid=(n // _R,),
        in_specs=[
            pl.BlockSpec((_R, d), lambda i: (i, 0)),
            pl.BlockSpec((_R, d), lambda i: (i, 0)),
            pl.BlockSpec((_R, d), lambda i: (i, 0)),
            pl.BlockSpec((_R, 1), lambda i: (i, 0)),
            pl.BlockSpec((1, 1), lambda i: (0, 0)),
        ],
        out_specs=pl.BlockSpec((_R, 1), lambda i: (i, 0)),
        out_shape=jax.ShapeDtypeStruct((n, 1), jnp.float32),
    )(p16, a0, a1, dis, b3)


def kernel(x, edge_index, W1, b1, W2, b2, W3, b3):
    n, d_feat = x.shape
    e = edge_index.shape[1]
    tpad = ((n + _R - 1) // _R) * _R
    assert tpad % (_NS * 8) == 0 and e % (_NW * _K) == 0

    nch = e // (_NW * _K)
    assert nch % _NB == 0
    src = edge_index[0].astype(jnp.int32).reshape(_NW, nch, _K)
    dst = edge_index[1].astype(jnp.int32).reshape(_NW, nch, _K)
    x_pad = jnp.zeros((tpad, d_feat), jnp.float32).at[:n].set(x)
    z1 = jnp.zeros((tpad,), jnp.float32)
    z64 = jnp.zeros((tpad, W1.shape[1]), jnp.float32)
    z32 = jnp.zeros((tpad, W2.shape[1]), jnp.float32)
    z16 = jnp.zeros((tpad, 16), jnp.float32)

    degp = _sc_degree(dst, z1, tpad)
    dg0 = degp[0].reshape(tpad, 1)
    dg1 = degp[1].reshape(tpad, 1)
    p1, dis = _tc_first(x_pad, W1, dg0, dg1)
    a = _sc_scatter(src, dst, p1, z64)
    p2 = _tc_mid(p1, a[0], a[1], dis, b1.reshape(1, -1), W2, W2.shape[1])
    a = _sc_scatter(src, dst, p2, z32)
    p3 = _tc_mid(p2, a[0], a[1], dis, b2.reshape(1, -1), W3, 16)
    a = _sc_scatter(src, dst, p3, z16)
    out = _tc_last(p3, a[0], a[1], dis, b3.reshape(1, 1))
    return out[:n]


# speedup vs baseline: 41.8072x; 1.0864x over previous
import functools

import jax
import jax.numpy as jnp
from jax import lax
from jax.experimental import pallas as pl
from jax.experimental.pallas import tpu as pltpu
from jax.experimental.pallas import tpu_sc as plsc

_NC, _NS = 2, 16
_NW = _NC * _NS
_K = 80
_NB = 5
_R = 1024


def _sc_degree(dst2d, tpad):
    nw, nch, k = dst2d.shape
    rows = tpad // _NS
    nwave = nch // _NB
    mesh = plsc.VectorSubcoreMesh(core_axis_name="c", subcore_axis_name="s", num_cores=_NC, num_subcores=_NS)

    @functools.partial(
        pl.kernel,
        out_type=(jax.ShapeDtypeStruct((tpad,), jnp.float32),
                  jax.ShapeDtypeStruct((tpad,), jnp.float32)),
        mesh=mesh,
        compiler_params=pltpu.CompilerParams(use_tc_tiling_on_sc=False),
        scratch_types=[
            pltpu.VMEM((nch, k), jnp.int32),
            pltpu.VMEM((k,), jnp.float32),
            pltpu.VMEM((k,), jnp.float32),
            pltpu.VMEM_SHARED((tpad,), jnp.float32),
            pltpu.SemaphoreType.DMA,
        ],
    )
    def deg_kernel(dst_hbm, out0_hbm, out1_hbm, idst2, ones_v, zv, acc_sh, ssem):
        c = lax.axis_index("c")
        s = lax.axis_index("s")
        w = s * _NC + c
        base = s * rows
        for i in range(k // 16):
            ones_v[pl.ds(i * 16, 16)] = jnp.full((16,), 1.0, jnp.float32)
            zv[pl.ds(i * 16, 16)] = jnp.zeros((16,), jnp.float32)
        for i in range(rows // k):
            pltpu.sync_copy(zv, acc_sh.at[pl.ds(base + i * k, k)])
        pltpu.sync_copy(dst_hbm.at[w], idst2)
        plsc.subcore_barrier()

        def wave(t, carry):
            j0 = t * _NB
            descs = [
                pltpu.async_copy(ones_v, acc_sh.at[idst2.at[j0 + b]], ssem, add=True)
                for b in range(_NB)
            ]
            for dsc in descs:
                dsc.wait()
            return carry

        lax.fori_loop(0, nwave, wave, 0)
        plsc.subcore_barrier()

        @pl.when(c == 0)
        def _():
            pltpu.sync_copy(acc_sh.at[pl.ds(base, rows)], out0_hbm.at[pl.ds(base, rows)])

        @pl.when(c == 1)
        def _():
            pltpu.sync_copy(acc_sh.at[pl.ds(base, rows)], out1_hbm.at[pl.ds(base, rows)])

    return deg_kernel(dst2d)


def _sc_scatter(src2d, dst2d, p):
    nw, nch, k = src2d.shape
    tpad, d = p.shape
    rows = tpad // _NS
    nwave = nch // _NB
    mesh = plsc.VectorSubcoreMesh(core_axis_name="c", subcore_axis_name="s", num_cores=_NC, num_subcores=_NS)

    @functools.partial(
        pl.kernel,
        out_type=(jax.ShapeDtypeStruct((tpad, d), jnp.float32),
                  jax.ShapeDtypeStruct((tpad, d), jnp.float32)),
        mesh=mesh,
        compiler_params=pltpu.CompilerParams(use_tc_tiling_on_sc=False),
        scratch_types=[
            pltpu.VMEM((nch, k), jnp.int32),
            pltpu.VMEM((nch, k), jnp.int32),
            pltpu.VMEM((_NB, k, d), jnp.float32),
            pltpu.VMEM((k, d), jnp.float32),
            pltpu.VMEM_SHARED((tpad, d), jnp.float32),
        ] + [pltpu.SemaphoreType.DMA] * _NB + [pltpu.SemaphoreType.DMA],
    )
    def scat_kernel(src_hbm, dst_hbm, p_hbm, out0_hbm, out1_hbm,
                    isrc2, idst2, rows_v, zt_v, acc_sh, *sems):
        gsems = sems[:_NB]
        ssem = sems[_NB]
        c = lax.axis_index("c")
        s = lax.axis_index("s")
        w = s * _NC + c
        base = s * rows
        for r in range(k):
            for j in range(d // 16):
                zt_v[r, pl.ds(j * 16, 16)] = jnp.zeros((16,), jnp.float32)
        for i in range(rows // k):
            pltpu.sync_copy(zt_v, acc_sh.at[pl.ds(base + i * k, k)])
        pltpu.sync_copy(src_hbm.at[w], isrc2)
        pltpu.sync_copy(dst_hbm.at[w], idst2)
        plsc.subcore_barrier()

        def wave(t, carry):
            j0 = t * _NB
            gds = [
                pltpu.async_copy(p_hbm.at[isrc2.at[j0 + b]], rows_v.at[b], gsems[b])
                for b in range(_NB)
            ]
            sds = []
            for b in range(_NB):
                gds[b].wait()
                sds.append(pltpu.async_copy(
                    rows_v.at[b], acc_sh.at[idst2.at[j0 + b]], ssem, add=True))
            for dsc in sds:
                dsc.wait()
            return carry

        lax.fori_loop(0, nwave, wave, 0)
        plsc.subcore_barrier()

        @pl.when(c == 0)
        def _():
            pltpu.sync_copy(acc_sh.at[pl.ds(base, rows)], out0_hbm.at[pl.ds(base, rows)])

        @pl.when(c == 1)
        def _():
            pltpu.sync_copy(acc_sh.at[pl.ds(base, rows)], out1_hbm.at[pl.ds(base, rows)])

    return scat_kernel(src2d, dst2d, p)


def _tc_matmul(x, w1):
    n, d_in = x.shape
    d_out = w1.shape[1]

    def body(x_ref, w_ref, h_ref):
        h_ref[...] = jnp.dot(x_ref[...], w_ref[...], preferred_element_type=jnp.float32)

    return pl.pallas_call(
        body,
        grid=(n // _R,),
        in_specs=[
            pl.BlockSpec((_R, d_in), lambda i: (i, 0)),
            pl.BlockSpec((d_in, d_out), lambda i: (0, 0)),
        ],
        out_specs=pl.BlockSpec((_R, d_out), lambda i: (i, 0)),
        out_shape=jax.ShapeDtypeStruct((n, d_out), jnp.float32),
    )(x, w1)


def _tc_scale(h1, dg0, dg1):
    n, d_out = h1.shape

    def body(h_ref, d0_ref, d1_ref, p_ref, dis_ref):
        deg = d0_ref[...] + d1_ref[...] + 1.0
        dis = lax.rsqrt(deg)
        p_ref[...] = h_ref[...] * dis
        dis_ref[...] = dis

    return pl.pallas_call(
        body,
        grid=(n // _R,),
        in_specs=[
            pl.BlockSpec((_R, d_out), lambda i: (i, 0)),
            pl.BlockSpec((_R, 1), lambda i: (i, 0)),
            pl.BlockSpec((_R, 1), lambda i: (i, 0)),
        ],
        out_specs=[
            pl.BlockSpec((_R, d_out), lambda i: (i, 0)),
            pl.BlockSpec((_R, 1), lambda i: (i, 0)),
        ],
        out_shape=[
            jax.ShapeDtypeStruct((n, d_out), jnp.float32),
            jax.ShapeDtypeStruct((n, 1), jnp.float32),
        ],
    )(h1, dg0, dg1)


def _tc_mid(p, a0, a1, dis, b, w, out_width):
    n, d = p.shape
    d2 = w.shape[1]

    def body(p_ref, a0_ref, a1_ref, dis_ref, b_ref, w_ref, o_ref):
        dis = dis_ref[...]
        hbar = (a0_ref[...] + a1_ref[...] + p_ref[...]) * dis + b_ref[...]
        h = jnp.maximum(hbar, 0.0)
        q = jnp.dot(h, w_ref[...], preferred_element_type=jnp.float32) * dis
        if out_width != d2:
            q = jnp.broadcast_to(q, (q.shape[0], out_width))
        o_ref[...] = q

    return pl.pallas_call(
        body,
        grid=(n // _R,),
        in_specs=[
            pl.BlockSpec((_R, d), lambda i: (i, 0)),
            pl.BlockSpec((_R, d), lambda i: (i, 0)),
            pl.BlockSpec((_R, d), lambda i: (i, 0)),
            pl.BlockSpec((_R, 1), lambda i: (i, 0)),
            pl.BlockSpec((1, d), lambda i: (0, 0)),
            pl.BlockSpec((d, d2), lambda i: (0, 0)),
        ],
        out_specs=pl.BlockSpec((_R, out_width), lambda i: (i, 0)),
        out_shape=jax.ShapeDtypeStruct((n, out_width), jnp.float32),
    )(p, a0, a1, dis, b, w)


def _tc_last(p16, a0, a1, dis, b3):
    n, d = p16.shape

    def body(p_ref, a0_ref, a1_ref, dis_ref, b_ref, o_ref):
        acc = a0_ref[...][:, :1] + a1_ref[...][:, :1] + p_ref[...][:, :1]
        v = acc * dis_ref[...] + b_ref[...]
        o_ref[...] = jax.nn.sigmoid(v)

    return pl.pallas_call(
        body,
        grid=(n // _R,),
        in_specs=[
            pl.BlockSpec((_R, d), lambda i: (i, 0)),
            pl.BlockSpec((_R, d), lambda i: (i, 0)),
            pl.BlockSpec((_R, d), lambda i: (i, 0)),
            pl.BlockSpec((_R, 1), lambda i: (i, 0)),
            pl.BlockSpec((1, 1), lambda i: (0, 0)),
        ],
        out_specs=pl.BlockSpec((_R, 1), lambda i: (i, 0)),
        out_shape=jax.ShapeDtypeStruct((n, 1), jnp.float32),
    )(p16, a0, a1, dis, b3)


def kernel(x, edge_index, W1, b1, W2, b2, W3, b3):
    n, d_feat = x.shape
    e = edge_index.shape[1]
    tpad = ((n + _R - 1) // _R) * _R
    assert tpad % (_NS * 8) == 0 and e % (_NW * _K) == 0

    nch = e // (_NW * _K)
    assert nch % _NB == 0
    src = edge_index[0].astype(jnp.int32).reshape(_NW, nch, _K)
    dst = edge_index[1].astype(jnp.int32).reshape(_NW, nch, _K)
    x_pad = jnp.zeros((tpad, d_feat), jnp.float32).at[:n].set(x)

    degp0, degp1 = _sc_degree(dst, tpad)
    dg0 = degp0.reshape(tpad, 1)
    dg1 = degp1.reshape(tpad, 1)
    h1 = _tc_matmul(x_pad, W1)
    p1, dis = _tc_scale(h1, dg0, dg1)
    a0, a1 = _sc_scatter(src, dst, p1)
    p2 = _tc_mid(p1, a0, a1, dis, b1.reshape(1, -1), W2, W2.shape[1])
    a0, a1 = _sc_scatter(src, dst, p2)
    p3 = _tc_mid(p2, a0, a1, dis, b2.reshape(1, -1), W3, 16)
    a0, a1 = _sc_scatter(src, dst, p3)
    out = _tc_last(p3, a0, a1, dis, b3.reshape(1, 1))
    return out[:n]


# speedup vs baseline: 46.0309x; 1.1010x over previous
import functools

import jax
import jax.numpy as jnp
from jax import lax
from jax.experimental import pallas as pl
from jax.experimental.pallas import tpu as pltpu
from jax.experimental.pallas import tpu_sc as plsc

_NC, _NS = 2, 16
_NW = _NC * _NS
_K = 80
_NB = 5
_R = 1024


def _sc_degree(dst2d, tpad):
    nw, nch, k = dst2d.shape
    rows = tpad // _NS
    nwave = nch // _NB
    mesh = plsc.VectorSubcoreMesh(core_axis_name="c", subcore_axis_name="s", num_cores=_NC, num_subcores=_NS)

    @functools.partial(
        pl.kernel,
        out_type=(jax.ShapeDtypeStruct((tpad,), jnp.float32),
                  jax.ShapeDtypeStruct((tpad,), jnp.float32)),
        mesh=mesh,
        compiler_params=pltpu.CompilerParams(use_tc_tiling_on_sc=False),
        scratch_types=[
            pltpu.VMEM((nch, k), jnp.int32),
            pltpu.VMEM((k,), jnp.float32),
            pltpu.VMEM((k,), jnp.float32),
            pltpu.VMEM_SHARED((tpad,), jnp.float32),
            pltpu.SemaphoreType.DMA,
        ],
    )
    def deg_kernel(dst_hbm, out0_hbm, out1_hbm, idst2, ones_v, zv, acc_sh, ssem):
        c = lax.axis_index("c")
        s = lax.axis_index("s")
        w = s * _NC + c
        base = s * rows
        for i in range(k // 16):
            ones_v[pl.ds(i * 16, 16)] = jnp.full((16,), 1.0, jnp.float32)
            zv[pl.ds(i * 16, 16)] = jnp.zeros((16,), jnp.float32)
        for i in range(rows // k):
            pltpu.sync_copy(zv, acc_sh.at[pl.ds(base + i * k, k)])
        pltpu.sync_copy(dst_hbm.at[w], idst2)
        plsc.subcore_barrier()

        def wave(t, carry):
            j0 = t * _NB
            descs = [
                pltpu.async_copy(ones_v, acc_sh.at[idst2.at[j0 + b]], ssem, add=True)
                for b in range(_NB)
            ]
            for dsc in descs:
                dsc.wait()
            return carry

        lax.fori_loop(0, nwave, wave, 0)
        plsc.subcore_barrier()

        @pl.when(c == 0)
        def _():
            pltpu.sync_copy(acc_sh.at[pl.ds(base, rows)], out0_hbm.at[pl.ds(base, rows)])

        @pl.when(c == 1)
        def _():
            pltpu.sync_copy(acc_sh.at[pl.ds(base, rows)], out1_hbm.at[pl.ds(base, rows)])

    return deg_kernel(dst2d)


def _sc_scatter(src2d, dst2d, p):
    nw, nch, k = src2d.shape
    tpad, d = p.shape
    rows = tpad // _NS
    nwave = nch // _NB
    mesh = plsc.VectorSubcoreMesh(core_axis_name="c", subcore_axis_name="s", num_cores=_NC, num_subcores=_NS)

    @functools.partial(
        pl.kernel,
        out_type=(jax.ShapeDtypeStruct((tpad, d), jnp.float32),
                  jax.ShapeDtypeStruct((tpad, d), jnp.float32)),
        mesh=mesh,
        compiler_params=pltpu.CompilerParams(use_tc_tiling_on_sc=False),
        scratch_types=[
            pltpu.VMEM((nch, k), jnp.int32),
            pltpu.VMEM((nch, k), jnp.int32),
            pltpu.VMEM((_NB, k, d), jnp.float32),
            pltpu.VMEM((k, d), jnp.float32),
            pltpu.VMEM_SHARED((tpad, d), jnp.float32),
        ] + [pltpu.SemaphoreType.DMA] * (2 * _NB),
    )
    def scat_kernel(src_hbm, dst_hbm, p_hbm, out0_hbm, out1_hbm,
                    isrc2, idst2, rows_v, zt_v, acc_sh, *sems):
        gsems = sems[:_NB]
        ssems = sems[_NB:]
        c = lax.axis_index("c")
        s = lax.axis_index("s")
        w = s * _NC + c
        base = s * rows
        for r in range(k):
            for j in range(d // 16):
                zt_v[r, pl.ds(j * 16, 16)] = jnp.zeros((16,), jnp.float32)
        for i in range(rows // k):
            pltpu.sync_copy(zt_v, acc_sh.at[pl.ds(base + i * k, k)])
        pltpu.sync_copy(src_hbm.at[w], isrc2)
        pltpu.sync_copy(dst_hbm.at[w], idst2)
        plsc.subcore_barrier()

        for b in range(_NB):
            pltpu.async_copy(p_hbm.at[isrc2.at[b]], rows_v.at[b], gsems[b])

        def wave(t, carry):
            j0 = t * _NB
            sds = []
            for b in range(_NB):
                pltpu.make_async_copy(
                    p_hbm.at[isrc2.at[j0 + b]], rows_v.at[b], gsems[b]).wait()
                sds.append(pltpu.async_copy(
                    rows_v.at[b], acc_sh.at[idst2.at[j0 + b]], ssems[b], add=True))
            for b in range(_NB):
                sds[b].wait()

                @pl.when(t < nwave - 1)
                def _(b=b, j0=j0):
                    pltpu.async_copy(
                        p_hbm.at[isrc2.at[j0 + _NB + b]], rows_v.at[b], gsems[b])
            return carry

        lax.fori_loop(0, nwave, wave, 0)
        plsc.subcore_barrier()

        @pl.when(c == 0)
        def _():
            pltpu.sync_copy(acc_sh.at[pl.ds(base, rows)], out0_hbm.at[pl.ds(base, rows)])

        @pl.when(c == 1)
        def _():
            pltpu.sync_copy(acc_sh.at[pl.ds(base, rows)], out1_hbm.at[pl.ds(base, rows)])

    return scat_kernel(src2d, dst2d, p)


def _tc_matmul(x, w1):
    n, d_in = x.shape
    d_out = w1.shape[1]

    def body(x_ref, w_ref, h_ref):
        h_ref[...] = jnp.dot(x_ref[...], w_ref[...], preferred_element_type=jnp.float32)

    return pl.pallas_call(
        body,
        grid=(n // _R,),
        in_specs=[
            pl.BlockSpec((_R, d_in), lambda i: (i, 0)),
            pl.BlockSpec((d_in, d_out), lambda i: (0, 0)),
        ],
        out_specs=pl.BlockSpec((_R, d_out), lambda i: (i, 0)),
        out_shape=jax.ShapeDtypeStruct((n, d_out), jnp.float32),
    )(x, w1)


def _tc_scale(h1, dg0, dg1):
    n, d_out = h1.shape

    def body(h_ref, d0_ref, d1_ref, p_ref, dis_ref):
        deg = d0_ref[...] + d1_ref[...] + 1.0
        dis = lax.rsqrt(deg)
        p_ref[...] = h_ref[...] * dis
        dis_ref[...] = dis

    return pl.pallas_call(
        body,
        grid=(n // _R,),
        in_specs=[
            pl.BlockSpec((_R, d_out), lambda i: (i, 0)),
            pl.BlockSpec((_R, 1), lambda i: (i, 0)),
            pl.BlockSpec((_R, 1), lambda i: (i, 0)),
        ],
        out_specs=[
            pl.BlockSpec((_R, d_out), lambda i: (i, 0)),
            pl.BlockSpec((_R, 1), lambda i: (i, 0)),
        ],
        out_shape=[
            jax.ShapeDtypeStruct((n, d_out), jnp.float32),
            jax.ShapeDtypeStruct((n, 1), jnp.float32),
        ],
    )(h1, dg0, dg1)


def _tc_mid(p, a0, a1, dis, b, w, out_width):
    n, d = p.shape
    d2 = w.shape[1]

    def body(p_ref, a0_ref, a1_ref, dis_ref, b_ref, w_ref, o_ref):
        dis = dis_ref[...]
        hbar = (a0_ref[...] + a1_ref[...] + p_ref[...]) * dis + b_ref[...]
        h = jnp.maximum(hbar, 0.0)
        q = jnp.dot(h, w_ref[...], preferred_element_type=jnp.float32) * dis
        if out_width != d2:
            q = jnp.broadcast_to(q, (q.shape[0], out_width))
        o_ref[...] = q

    return pl.pallas_call(
        body,
        grid=(n // _R,),
        in_specs=[
            pl.BlockSpec((_R, d), lambda i: (i, 0)),
            pl.BlockSpec((_R, d), lambda i: (i, 0)),
            pl.BlockSpec((_R, d), lambda i: (i, 0)),
            pl.BlockSpec((_R, 1), lambda i: (i, 0)),
            pl.BlockSpec((1, d), lambda i: (0, 0)),
            pl.BlockSpec((d, d2), lambda i: (0, 0)),
        ],
        out_specs=pl.BlockSpec((_R, out_width), lambda i: (i, 0)),
        out_shape=jax.ShapeDtypeStruct((n, out_width), jnp.float32),
    )(p, a0, a1, dis, b, w)


def _tc_last(p16, a0, a1, dis, b3):
    n, d = p16.shape

    def body(p_ref, a0_ref, a1_ref, dis_ref, b_ref, o_ref):
        acc = a0_ref[...][:, :1] + a1_ref[...][:, :1] + p_ref[...][:, :1]
        v = acc * dis_ref[...] + b_ref[...]
        o_ref[...] = jax.nn.sigmoid(v)

    return pl.pallas_call(
        body,
        grid=(n // _R,),
        in_specs=[
            pl.BlockSpec((_R, d), lambda i: (i, 0)),
            pl.BlockSpec((_R, d), lambda i: (i, 0)),
            pl.BlockSpec((_R, d), lambda i: (i, 0)),
            pl.BlockSpec((_R, 1), lambda i: (i, 0)),
            pl.BlockSpec((1, 1), lambda i: (0, 0)),
        ],
        out_specs=pl.BlockSpec((_R, 1), lambda i: (i, 0)),
        out_shape=jax.ShapeDtypeStruct((n, 1), jnp.float32),
    )(p16, a0, a1, dis, b3)


def kernel(x, edge_index, W1, b1, W2, b2, W3, b3):
    n, d_feat = x.shape
    e = edge_index.shape[1]
    tpad = ((n + _R - 1) // _R) * _R
    assert tpad % (_NS * 8) == 0 and e % (_NW * _K) == 0

    nch = e // (_NW * _K)
    assert nch % _NB == 0
    src = edge_index[0].astype(jnp.int32).reshape(_NW, nch, _K)
    dst = edge_index[1].astype(jnp.int32).reshape(_NW, nch, _K)
    x_pad = jnp.zeros((tpad, d_feat), jnp.float32).at[:n].set(x)

    degp0, degp1 = _sc_degree(dst, tpad)
    dg0 = degp0.reshape(tpad, 1)
    dg1 = degp1.reshape(tpad, 1)
    h1 = _tc_matmul(x_pad, W1)
    p1, dis = _tc_scale(h1, dg0, dg1)
    a0, a1 = _sc_scatter(src, dst, p1)
    p2 = _tc_mid(p1, a0, a1, dis, b1.reshape(1, -1), W2, W2.shape[1])
    a0, a1 = _sc_scatter(src, dst, p2)
    p3 = _tc_mid(p2, a0, a1, dis, b2.reshape(1, -1), W3, 16)
    a0, a1 = _sc_scatter(src, dst, p3)
    out = _tc_last(p3, a0, a1, dis, b3.reshape(1, 1))
    return out[:n]
